# baseline (speedup 1.0000x reference)
import jax
import jax.numpy as jnp
from jax import lax
from jax.experimental import pallas as pl
from jax.experimental.pallas import tpu as pltpu

EMBED = 32
UF = 16
IF = 16


_LANES = 128


def _gather_gru_body(uid_ref, iid_ref, inp_ref, uW_ref, ubih_ref, ubhh_ref,
                     iW_ref, ibih_ref, ibhh_ref, pu_ref, pi_ref,
                     newuT_ref, newiT_ref, out_ref,
                     umscr, imscr, uall, iall, semu, semi):
    B = inp_ref.shape[0]

    def issue(b, c):
        u = uid_ref[b]
        it = iid_ref[b]
        t = b // _LANES
        lt = pl.multiple_of(t * _LANES, _LANES)
        pltpu.make_async_copy(pu_ref.at[pl.ds(u, 1), :, pl.ds(lt, _LANES)],
                              umscr.at[pl.ds(b, 1)], semu.at[t]).start()
        pltpu.make_async_copy(pi_ref.at[pl.ds(it, 1), :, pl.ds(lt, _LANES)],
                              imscr.at[pl.ds(b, 1)], semi.at[t]).start()
        return c

    lax.fori_loop(0, B, issue, 0, unroll=8)

    sel = lax.broadcasted_iota(jnp.int32, (_LANES, EMBED, _LANES), 0)
    lane = lax.broadcasted_iota(jnp.int32, (_LANES, EMBED, _LANES), 2)
    diag = sel == lane

    for t in range(B // _LANES):
        src = pu_ref.at[pl.ds(0, _LANES), :, pl.ds(0, _LANES)]
        lo = t * _LANES
        pltpu.make_async_copy(src, umscr.at[pl.ds(lo, _LANES)],
                              semu.at[t]).wait()
        pltpu.make_async_copy(src, imscr.at[pl.ds(lo, _LANES)],
                              semi.at[t]).wait()
        uall[pl.ds(lo, _LANES), :] = jnp.sum(
            jnp.where(diag, umscr[lo:lo + _LANES], 0.0), axis=2)
        iall[pl.ds(lo, _LANES), :] = jnp.sum(
            jnp.where(diag, imscr[lo:lo + _LANES], 0.0), axis=2)

    um = uall[...]
    im = iall[...]
    inp = inp_ref[...]
    uf = inp[:, 2:2 + UF]
    itf = inp[:, 2 + UF:2 + UF + IF]
    xu = jnp.concatenate([um, uf, im, itf], axis=1)
    xi = jnp.concatenate([im, itf, um, uf], axis=1)

    def cell(x, W, bih, bhh):
        g = lax.dot_general(x, W, (((1,), (1,)), ((), ())),
                            preferred_element_type=jnp.float32)
        g = g + bih
        r = jax.nn.sigmoid(g[:, :EMBED] + bhh[:, :EMBED])
        z = jax.nn.sigmoid(g[:, EMBED:2 * EMBED] + bhh[:, EMBED:2 * EMBED])
        n = jnp.tanh(g[:, 2 * EMBED:] + r * bhh[:, 2 * EMBED:])
        h = (1.0 - z) * n
        norm = jnp.maximum(jnp.sqrt(jnp.sum(h * h, axis=1, keepdims=True)), 1e-12)
        return h / norm

    nu = cell(xu, uW_ref[...], ubih_ref[...], ubhh_ref[...])
    ni = cell(xi, iW_ref[...], ibih_ref[...], ibhh_ref[...])
    newuT_ref[...] = nu.T
    newiT_ref[...] = ni.T
    out_ref[...] = jnp.concatenate([inp[:, :2], nu, ni], axis=1)


def _gather_gru(uid, iid, inputs, uW_ih, ub_ih, ub_hh, iW_ih, ib_ih, ib_hh,
                pu, pi):
    B = inputs.shape[0]
    f32 = jnp.float32
    smem = pl.BlockSpec(memory_space=pltpu.MemorySpace.SMEM)
    hbm = pl.BlockSpec(memory_space=pltpu.MemorySpace.HBM)
    vmem = pl.BlockSpec(memory_space=pltpu.MemorySpace.VMEM)
    return pl.pallas_call(
        _gather_gru_body,
        in_specs=[smem, smem, vmem, vmem, vmem, vmem, vmem, vmem, vmem,
                  hbm, hbm],
        out_shape=(
            jax.ShapeDtypeStruct((EMBED, B), f32),
            jax.ShapeDtypeStruct((EMBED, B), f32),
            jax.ShapeDtypeStruct((B, 2 + 2 * EMBED), f32),
        ),
        scratch_shapes=[
            pltpu.VMEM((B, EMBED, _LANES), f32),
            pltpu.VMEM((B, EMBED, _LANES), f32),
            pltpu.VMEM((B, EMBED), f32),
            pltpu.VMEM((B, EMBED), f32),
            pltpu.SemaphoreType.DMA((B // _LANES,)),
            pltpu.SemaphoreType.DMA((B // _LANES,)),
        ],
    )(uid, iid, inputs,
      uW_ih, ub_ih.reshape(1, -1), ub_hh.reshape(1, -1),
      iW_ih, ib_ih.reshape(1, -1), ib_hh.reshape(1, -1),
      pu, pi)


_UBLK = 40


def _select_body(uids_ref, iids_ref, newuT_ref, newiT_ref, pu_ref, pi_ref,
                 uout_ref, iout_ref):
    i = pl.program_id(0)
    u0 = i * _UBLK
    urow = lax.broadcasted_iota(jnp.int32, (_UBLK, 1, 1), 0) + u0
    umask = urow == uids_ref[...].reshape(1, 1, -1)
    imask = urow == iids_ref[...].reshape(1, 1, -1)
    uout_ref[...] = jnp.where(umask, newuT_ref[...][None], pu_ref[...])
    iout_ref[...] = jnp.where(imask, newiT_ref[...][None], pi_ref[...])


def _select_scatter(uids, iids, newuT, newiT, pu, pi):
    N, E, B = pu.shape
    grid = (N // _UBLK,)
    const2 = lambda i: (0, 0)
    const3 = lambda i: (i, 0, 0)
    bank_spec = pl.BlockSpec((_UBLK, E, B), const3)
    return pl.pallas_call(
        _select_body,
        grid=grid,
        in_specs=[
            pl.BlockSpec((1, B), const2),
            pl.BlockSpec((1, B), const2),
            pl.BlockSpec((E, B), const2),
            pl.BlockSpec((E, B), const2),
            bank_spec,
            bank_spec,
        ],
        out_specs=(bank_spec, bank_spec),
        out_shape=(jax.ShapeDtypeStruct((N, E, B), pu.dtype),
                   jax.ShapeDtypeStruct((N, E, B), pi.dtype)),
    )(uids.reshape(1, B), iids.reshape(1, B), newuT, newiT, pu, pi)


def kernel(inputs, user_memory, item_memory, uW_ih, uW_hh, ub_ih, ub_hh,
           iW_ih, iW_hh, ib_ih, ib_hh):
    B = inputs.shape[0]
    uid = inputs[:, 0].astype(jnp.int32)
    iid = inputs[:, 1].astype(jnp.int32)
    pu = jnp.transpose(user_memory, (1, 2, 0))
    pi = jnp.transpose(item_memory, (1, 2, 0))

    newuT, newiT, out = _gather_gru(uid, iid, inputs, uW_ih, ub_ih, ub_hh,
                                    iW_ih, ib_ih, ib_hh, pu, pi)
    pu_new, pi_new = _select_scatter(uid, iid, newuT, newiT, pu, pi)
    new_user_memory = jnp.transpose(pu_new, (2, 0, 1))
    new_item_memory = jnp.transpose(pi_new, (2, 0, 1))
    return (out, new_user_memory, new_item_memory)


# speedup vs baseline: 1.0027x; 1.0027x over previous
import jax
import jax.numpy as jnp
from jax import lax
from jax.experimental import pallas as pl
from jax.experimental.pallas import tpu as pltpu

EMBED = 32
UF = 16
IF = 16


_LANES = 128


def _gather_gru_body(uid_ref, iid_ref, inp_ref, uW_ref, ubih_ref, ubhh_ref,
                     iW_ref, ibih_ref, ibhh_ref, pu_ref, pi_ref,
                     newuT_ref, newiT_ref, out_ref,
                     umscr, imscr, uall, iall, semu, semi):
    B = inp_ref.shape[0]

    def issue(b, c):
        u = uid_ref[b]
        it = iid_ref[b]
        t = b // _LANES
        lt = pl.multiple_of(t * _LANES, _LANES)
        pltpu.make_async_copy(pu_ref.at[pl.ds(u, 1), :, pl.ds(lt, _LANES)],
                              umscr.at[pl.ds(b, 1)], semu.at[t]).start()
        pltpu.make_async_copy(pi_ref.at[pl.ds(it, 1), :, pl.ds(lt, _LANES)],
                              imscr.at[pl.ds(b, 1)], semi.at[t]).start()
        return c

    lax.fori_loop(0, B, issue, 0, unroll=8)

    sel = lax.broadcasted_iota(jnp.int32, (_LANES, EMBED, _LANES), 0)
    lane = lax.broadcasted_iota(jnp.int32, (_LANES, EMBED, _LANES), 2)
    diag = sel == lane

    for t in range(B // _LANES):
        src = pu_ref.at[pl.ds(0, _LANES), :, pl.ds(0, _LANES)]
        lo = t * _LANES
        pltpu.make_async_copy(src, umscr.at[pl.ds(lo, _LANES)],
                              semu.at[t]).wait()
        pltpu.make_async_copy(src, imscr.at[pl.ds(lo, _LANES)],
                              semi.at[t]).wait()
        uall[pl.ds(lo, _LANES), :] = jnp.sum(
            jnp.where(diag, umscr[lo:lo + _LANES], 0.0), axis=2)
        iall[pl.ds(lo, _LANES), :] = jnp.sum(
            jnp.where(diag, imscr[lo:lo + _LANES], 0.0), axis=2)

    um = uall[...]
    im = iall[...]
    inp = inp_ref[...]
    uf = inp[:, 2:2 + UF]
    itf = inp[:, 2 + UF:2 + UF + IF]
    xu = jnp.concatenate([um, uf, im, itf], axis=1)
    xi = jnp.concatenate([im, itf, um, uf], axis=1)

    def cell(x, W, bih, bhh):
        g = lax.dot_general(x, W, (((1,), (1,)), ((), ())),
                            preferred_element_type=jnp.float32)
        g = g + bih
        r = jax.nn.sigmoid(g[:, :EMBED] + bhh[:, :EMBED])
        z = jax.nn.sigmoid(g[:, EMBED:2 * EMBED] + bhh[:, EMBED:2 * EMBED])
        n = jnp.tanh(g[:, 2 * EMBED:] + r * bhh[:, 2 * EMBED:])
        h = (1.0 - z) * n
        norm = jnp.maximum(jnp.sqrt(jnp.sum(h * h, axis=1, keepdims=True)), 1e-12)
        return h / norm

    nu = cell(xu, uW_ref[...], ubih_ref[...], ubhh_ref[...])
    ni = cell(xi, iW_ref[...], ibih_ref[...], ibhh_ref[...])
    newuT_ref[...] = nu.T
    newiT_ref[...] = ni.T
    out_ref[...] = jnp.concatenate([inp[:, :2], nu, ni], axis=1)


def _gather_gru(uid, iid, inputs, uW_ih, ub_ih, ub_hh, iW_ih, ib_ih, ib_hh,
                pu, pi):
    B = inputs.shape[0]
    f32 = jnp.float32
    smem = pl.BlockSpec(memory_space=pltpu.MemorySpace.SMEM)
    hbm = pl.BlockSpec(memory_space=pltpu.MemorySpace.HBM)
    vmem = pl.BlockSpec(memory_space=pltpu.MemorySpace.VMEM)
    return pl.pallas_call(
        _gather_gru_body,
        in_specs=[smem, smem, vmem, vmem, vmem, vmem, vmem, vmem, vmem,
                  hbm, hbm],
        out_shape=(
            jax.ShapeDtypeStruct((EMBED, B), f32),
            jax.ShapeDtypeStruct((EMBED, B), f32),
            jax.ShapeDtypeStruct((B, 2 + 2 * EMBED), f32),
        ),
        scratch_shapes=[
            pltpu.VMEM((B, EMBED, _LANES), f32),
            pltpu.VMEM((B, EMBED, _LANES), f32),
            pltpu.VMEM((B, EMBED), f32),
            pltpu.VMEM((B, EMBED), f32),
            pltpu.SemaphoreType.DMA((B // _LANES,)),
            pltpu.SemaphoreType.DMA((B // _LANES,)),
        ],
    )(uid, iid, inputs,
      uW_ih, ub_ih.reshape(1, -1), ub_hh.reshape(1, -1),
      iW_ih, ib_ih.reshape(1, -1), ib_hh.reshape(1, -1),
      pu, pi)


_UBLK = 50


def _select_body(uids_ref, iids_ref, newuT_ref, newiT_ref, pu_ref, pi_ref,
                 uout_ref, iout_ref):
    i = pl.program_id(0)
    u0 = i * _UBLK
    urow = lax.broadcasted_iota(jnp.int32, (_UBLK, 1, 1), 0) + u0
    umask = urow == uids_ref[...].reshape(1, 1, -1)
    imask = urow == iids_ref[...].reshape(1, 1, -1)
    uout_ref[...] = jnp.where(umask, newuT_ref[...][None], pu_ref[...])
    iout_ref[...] = jnp.where(imask, newiT_ref[...][None], pi_ref[...])


def _select_scatter(uids, iids, newuT, newiT, pu, pi):
    N, E, B = pu.shape
    grid = (N // _UBLK,)
    const2 = lambda i: (0, 0)
    const3 = lambda i: (i, 0, 0)
    bank_spec = pl.BlockSpec((_UBLK, E, B), const3)
    return pl.pallas_call(
        _select_body,
        grid=grid,
        in_specs=[
            pl.BlockSpec((1, B), const2),
            pl.BlockSpec((1, B), const2),
            pl.BlockSpec((E, B), const2),
            pl.BlockSpec((E, B), const2),
            bank_spec,
            bank_spec,
        ],
        out_specs=(bank_spec, bank_spec),
        out_shape=(jax.ShapeDtypeStruct((N, E, B), pu.dtype),
                   jax.ShapeDtypeStruct((N, E, B), pi.dtype)),
    )(uids.reshape(1, B), iids.reshape(1, B), newuT, newiT, pu, pi)


def kernel(inputs, user_memory, item_memory, uW_ih, uW_hh, ub_ih, ub_hh,
           iW_ih, iW_hh, ib_ih, ib_hh):
    B = inputs.shape[0]
    uid = inputs[:, 0].astype(jnp.int32)
    iid = inputs[:, 1].astype(jnp.int32)
    pu = jnp.transpose(user_memory, (1, 2, 0))
    pi = jnp.transpose(item_memory, (1, 2, 0))

    newuT, newiT, out = _gather_gru(uid, iid, inputs, uW_ih, ub_ih, ub_hh,
                                    iW_ih, ib_ih, ib_hh, pu, pi)
    pu_new, pi_new = _select_scatter(uid, iid, newuT, newiT, pu, pi)
    new_user_memory = jnp.transpose(pu_new, (2, 0, 1))
    new_item_memory = jnp.transpose(pi_new, (2, 0, 1))
    return (out, new_user_memory, new_item_memory)


# speedup vs baseline: 1.0116x; 1.0089x over previous
import jax
import jax.numpy as jnp
from jax import lax
from jax.experimental import pallas as pl
from jax.experimental.pallas import tpu as pltpu

EMBED = 32
UF = 16
IF = 16


_LANES = 128


def _gather_gru_body(uid_ref, iid_ref, inp_ref, uW_ref, ubih_ref, ubhh_ref,
                     iW_ref, ibih_ref, ibhh_ref, pu_ref, pi_ref,
                     newuT_ref, newiT_ref, out_ref,
                     umscr, imscr, uall, iall, semu, semi):
    B = inp_ref.shape[0]

    def issue(b, c):
        u = uid_ref[b]
        it = iid_ref[b]
        t = b // _LANES
        lt = pl.multiple_of(t * _LANES, _LANES)
        pltpu.make_async_copy(pu_ref.at[pl.ds(u, 1), :, pl.ds(lt, _LANES)],
                              umscr.at[pl.ds(b, 1)], semu.at[t]).start()
        pltpu.make_async_copy(pi_ref.at[pl.ds(it, 1), :, pl.ds(lt, _LANES)],
                              imscr.at[pl.ds(b, 1)], semi.at[t]).start()
        return c

    lax.fori_loop(0, B, issue, 0, unroll=8)

    sel = lax.broadcasted_iota(jnp.int32, (_LANES, EMBED, _LANES), 0)
    lane = lax.broadcasted_iota(jnp.int32, (_LANES, EMBED, _LANES), 2)
    diag = sel == lane

    for t in range(B // _LANES):
        src = pu_ref.at[pl.ds(0, _LANES), :, pl.ds(0, _LANES)]
        lo = t * _LANES
        pltpu.make_async_copy(src, umscr.at[pl.ds(lo, _LANES)],
                              semu.at[t]).wait()
        pltpu.make_async_copy(src, imscr.at[pl.ds(lo, _LANES)],
                              semi.at[t]).wait()
        uall[pl.ds(lo, _LANES), :] = jnp.sum(
            jnp.where(diag, umscr[lo:lo + _LANES], 0.0), axis=2)
        iall[pl.ds(lo, _LANES), :] = jnp.sum(
            jnp.where(diag, imscr[lo:lo + _LANES], 0.0), axis=2)

    um = uall[...]
    im = iall[...]
    inp = inp_ref[...]
    uf = inp[:, 2:2 + UF]
    itf = inp[:, 2 + UF:2 + UF + IF]
    xu = jnp.concatenate([um, uf, im, itf], axis=1)
    xi = jnp.concatenate([im, itf, um, uf], axis=1)

    def cell(x, W, bih, bhh):
        g = lax.dot_general(x, W, (((1,), (1,)), ((), ())),
                            preferred_element_type=jnp.float32)
        g = g + bih
        r = jax.nn.sigmoid(g[:, :EMBED] + bhh[:, :EMBED])
        z = jax.nn.sigmoid(g[:, EMBED:2 * EMBED] + bhh[:, EMBED:2 * EMBED])
        n = jnp.tanh(g[:, 2 * EMBED:] + r * bhh[:, 2 * EMBED:])
        h = (1.0 - z) * n
        norm = jnp.maximum(jnp.sqrt(jnp.sum(h * h, axis=1, keepdims=True)), 1e-12)
        return h / norm

    nu = cell(xu, uW_ref[...], ubih_ref[...], ubhh_ref[...])
    ni = cell(xi, iW_ref[...], ibih_ref[...], ibhh_ref[...])
    newuT_ref[...] = nu.T
    newiT_ref[...] = ni.T
    out_ref[...] = jnp.concatenate([inp[:, :2], nu, ni], axis=1)


def _gather_gru(uid, iid, inputs, uW_ih, ub_ih, ub_hh, iW_ih, ib_ih, ib_hh,
                pu, pi):
    B = inputs.shape[0]
    f32 = jnp.float32
    smem = pl.BlockSpec(memory_space=pltpu.MemorySpace.SMEM)
    hbm = pl.BlockSpec(memory_space=pltpu.MemorySpace.HBM)
    vmem = pl.BlockSpec(memory_space=pltpu.MemorySpace.VMEM)
    return pl.pallas_call(
        _gather_gru_body,
        in_specs=[smem, smem, vmem, vmem, vmem, vmem, vmem, vmem, vmem,
                  hbm, hbm],
        out_shape=(
            jax.ShapeDtypeStruct((EMBED, B), f32),
            jax.ShapeDtypeStruct((EMBED, B), f32),
            jax.ShapeDtypeStruct((B, 2 + 2 * EMBED), f32),
        ),
        scratch_shapes=[
            pltpu.VMEM((B, EMBED, _LANES), f32),
            pltpu.VMEM((B, EMBED, _LANES), f32),
            pltpu.VMEM((B, EMBED), f32),
            pltpu.VMEM((B, EMBED), f32),
            pltpu.SemaphoreType.DMA((B // _LANES,)),
            pltpu.SemaphoreType.DMA((B // _LANES,)),
        ],
    )(uid, iid, inputs,
      uW_ih, ub_ih.reshape(1, -1), ub_hh.reshape(1, -1),
      iW_ih, ib_ih.reshape(1, -1), ib_hh.reshape(1, -1),
      pu, pi)


_UBLK = 100


def _select_body(ids_ref, newT_ref, p_ref, out_ref):
    i = pl.program_id(0)
    u0 = i * _UBLK
    urow = lax.broadcasted_iota(jnp.int32, (_UBLK, 1, 1), 0) + u0
    mask = urow == ids_ref[...].reshape(1, 1, -1)
    out_ref[...] = jnp.where(mask, newT_ref[...][None], p_ref[...])


def _select_scatter(ids, newT, p):
    N, E, B = p.shape
    grid = (N // _UBLK,)
    const2 = lambda i: (0, 0)
    const3 = lambda i: (i, 0, 0)
    bank_spec = pl.BlockSpec((_UBLK, E, B), const3)
    return pl.pallas_call(
        _select_body,
        grid=grid,
        in_specs=[
            pl.BlockSpec((1, B), const2),
            pl.BlockSpec((E, B), const2),
            bank_spec,
        ],
        out_specs=bank_spec,
        out_shape=jax.ShapeDtypeStruct((N, E, B), p.dtype),
    )(ids.reshape(1, B), newT, p)


def kernel(inputs, user_memory, item_memory, uW_ih, uW_hh, ub_ih, ub_hh,
           iW_ih, iW_hh, ib_ih, ib_hh):
    B = inputs.shape[0]
    uid = inputs[:, 0].astype(jnp.int32)
    iid = inputs[:, 1].astype(jnp.int32)
    pu = jnp.transpose(user_memory, (1, 2, 0))
    pi = jnp.transpose(item_memory, (1, 2, 0))

    newuT, newiT, out = _gather_gru(uid, iid, inputs, uW_ih, ub_ih, ub_hh,
                                    iW_ih, ib_ih, ib_hh, pu, pi)
    pu_new = _select_scatter(uid, newuT, pu)
    pi_new = _select_scatter(iid, newiT, pi)
    new_user_memory = jnp.transpose(pu_new, (2, 0, 1))
    new_item_memory = jnp.transpose(pi_new, (2, 0, 1))
    return (out, new_user_memory, new_item_memory)
